# baseline (speedup 1.0000x reference)
import jax
import jax.numpy as jnp
from jax import lax
from jax.experimental import pallas as pl
from jax.experimental.pallas import tpu as pltpu
from jax.experimental.pallas import tpu_sc as plsc

_D = 768
_E = 8
_H = 3072
_T = 2048
_CAP = 640
_EC = _E * _CAP
_NW = 32
_ROWS_PER_W = _EC // _NW
_GCHUNK = 80
_TOK_PER_W = _T // _NW
_HC = 1
_HB = _H // _HC


def _router_body(x_ref, wg_ref, sidx_ref, cidx_ref, wflat_ref, lb_ref, zz_ref,
                 x16_ref):
    x = x_ref[...]
    xi = pltpu.bitcast(x, jnp.int32)
    rb = xi + 0x7FFF + ((xi >> 16) & 1)
    lo = (rb[:, : _D // 2] >> 16) & 0xFFFF
    hi = rb[:, _D // 2 :] & jnp.int32(-65536)
    x16_ref[...] = lo | hi
    wg = wg_ref[...]
    logits = jnp.dot(x, wg, preferred_element_type=jnp.float32)

    ei = lax.broadcasted_iota(jnp.int32, (_T, _E), 1)
    m1 = jnp.max(logits, axis=1, keepdims=True)
    a1 = jnp.min(jnp.where(logits == m1, ei, _E), axis=1, keepdims=True)
    l2 = jnp.where(ei == a1, -jnp.inf, logits)
    m2 = jnp.max(l2, axis=1, keepdims=True)
    a2 = jnp.min(jnp.where(l2 == m2, ei, _E), axis=1, keepdims=True)

    d = jnp.exp(m2 - m1)
    w1 = 1.0 / (1.0 + d)
    w2 = d / (1.0 + d)

    oh1 = (ei == a1).astype(jnp.float32)
    oh2 = (ei == a2).astype(jnp.float32)
    oh = jnp.concatenate([oh1, oh2], axis=0)
    c = oh
    s = 1
    while s < 2 * _T:
        pad = jnp.zeros((s, _E), jnp.float32)
        c = c + jnp.concatenate([pad, c[: 2 * _T - s]], axis=0)
        s *= 2
    rank = jnp.sum(oh * c, axis=1, keepdims=True) - 1.0
    ranki = rank.astype(jnp.int32)

    e_flat = jnp.concatenate([a1, a2], axis=0)
    w_flat = jnp.concatenate([w1, w2], axis=0)
    valid = (ranki < _CAP) & (w_flat > 0.0)
    slot = e_flat * _CAP + ranki
    ji = lax.broadcasted_iota(jnp.int32, (2 * _T, 1), 0)
    sidx_ref[...] = jnp.where(valid, slot, _EC + (ji & 7))

    counts = c[2 * _T - 1 : 2 * _T, :]
    ei_row = lax.broadcasted_iota(jnp.int32, (1, _E), 1)
    cmax = jnp.max(counts, axis=1, keepdims=True)
    emax = jnp.min(jnp.where(counts == cmax, ei_row, _E), axis=1, keepdims=True)
    dummy = emax * _CAP
    cidx_ref[...] = jnp.where(valid, slot, dummy)
    wflat_ref[...] = jnp.where(valid, w_flat, 0.0)

    pe = jnp.exp(logits - m1)
    se = jnp.sum(pe, axis=1, keepdims=True)
    probs = pe / se
    ppe = jnp.sum(probs, axis=0, keepdims=True) * (1.0 / _T)
    v1 = valid[:_T].astype(jnp.float32)
    v2 = valid[_T:].astype(jnp.float32)
    tpe = jnp.sum(oh1 * v1 + oh2 * v2, axis=0, keepdims=True) * (1.0 / _T)
    lb_ref[...] = _E * jnp.sum(ppe * tpe, axis=1, keepdims=True)
    z = m1 + jnp.log(se)
    zz_ref[...] = jnp.sum(z * z, axis=0, keepdims=True) * (1.0 / _T)


def _dispatch_body(sidx_hbm, x_hbm, eb_hbm, i0_v, i1_v, rows_v, semg, sem0, sem1):
    cid = lax.axis_index("c")
    sid = lax.axis_index("s")
    wid = sid * 2 + cid
    base = wid * _TOK_PER_W

    g = pltpu.async_copy(x_hbm.at[pl.ds(base, _TOK_PER_W)], rows_v, semg)
    pltpu.sync_copy(sidx_hbm.at[pl.ds(base, _TOK_PER_W)], i0_v)
    pltpu.sync_copy(sidx_hbm.at[pl.ds(_T + base, _TOK_PER_W)], i1_v)
    g.wait()
    c0 = pltpu.async_copy(rows_v, eb_hbm.at[i0_v], sem0)
    c1 = pltpu.async_copy(rows_v, eb_hbm.at[i1_v], sem1)
    c0.wait()
    c1.wait()


def _mlp_body(eb_ref, wfc_ref, wp_ref, out_ref):
    hc = pl.program_id(1)
    ebw = eb_ref[...]
    eb_lo = pltpu.bitcast(ebw << 16, jnp.float32).astype(jnp.bfloat16)
    eb_hi = pltpu.bitcast(ebw & jnp.int32(-65536), jnp.float32).astype(jnp.bfloat16)
    wfc16 = wfc_ref[0].astype(jnp.bfloat16)
    h = jnp.dot(eb_lo, wfc16[: _D // 2],
                preferred_element_type=jnp.float32)
    h = h + jnp.dot(eb_hi, wfc16[_D // 2 :],
                    preferred_element_type=jnp.float32)
    h = 0.5 * h * (1.0 + lax.erf(h * 0.7071067811865476))
    part = jnp.dot(h.astype(jnp.bfloat16), wp_ref[0].astype(jnp.bfloat16),
                   preferred_element_type=jnp.float32)

    @pl.when(hc == 0)
    def _():
        out_ref[...] = part

    @pl.when(hc > 0)
    def _():
        out_ref[...] = out_ref[...] + part


def _combine_body(cidx_hbm, wf_hbm, eo_hbm, y_hbm,
                  i0_v, i1_v, w0_v, w1_v, b0, b1, sem0, sem1):
    cid = lax.axis_index("c")
    sid = lax.axis_index("s")
    wid = sid * 2 + cid
    base = wid * _TOK_PER_W

    pltpu.sync_copy(cidx_hbm.at[pl.ds(base, _TOK_PER_W)], i0_v)
    pltpu.sync_copy(cidx_hbm.at[pl.ds(_T + base, _TOK_PER_W)], i1_v)
    pltpu.sync_copy(wf_hbm.at[pl.ds(base, _TOK_PER_W)], w0_v)
    pltpu.sync_copy(wf_hbm.at[pl.ds(_T + base, _TOK_PER_W)], w1_v)

    hw = _TOK_PER_W // 2
    copies = []
    for half in range(2):
        r0 = half * hw
        copies.append(pltpu.async_copy(
            eo_hbm.at[i0_v.at[pl.ds(r0, hw)]], b0.at[pl.ds(r0, hw)], sem0))
        copies.append(pltpu.async_copy(
            eo_hbm.at[i1_v.at[pl.ds(r0, hw)]], b1.at[pl.ds(r0, hw)], sem1))

    def addrow(r, carry):
        rv = jnp.zeros((16,), jnp.int32) + r
        w0 = plsc.load_gather(w0_v, [rv])
        w1 = plsc.load_gather(w1_v, [rv])
        for cc in range(_D // 16):
            sl = pl.ds(cc * 16, 16)
            b0[r, sl] = b0[r, sl] * w0 + b1[r, sl] * w1
        return carry

    for half in range(2):
        copies[2 * half].wait()
        copies[2 * half + 1].wait()
        r0 = half * hw
        lax.fori_loop(r0, r0 + hw, addrow, 0)
        pltpu.sync_copy(b0.at[pl.ds(r0, hw)],
                        y_hbm.at[pl.ds(base + r0, hw)])


def kernel(x, w_gate, w_fc, w_proj):
    xf = x.reshape(_T, _D)

    sidx, cidx, wflat, lb, zz, x16 = pl.pallas_call(
        _router_body,
        out_shape=[
            jax.ShapeDtypeStruct((2 * _T, 1), jnp.int32),
            jax.ShapeDtypeStruct((2 * _T, 1), jnp.int32),
            jax.ShapeDtypeStruct((2 * _T, 1), jnp.float32),
            jax.ShapeDtypeStruct((1, 1), jnp.float32),
            jax.ShapeDtypeStruct((1, 1), jnp.float32),
            jax.ShapeDtypeStruct((_T, _D // 2), jnp.int32),
        ],
    )(xf, w_gate)

    mesh = plsc.VectorSubcoreMesh(core_axis_name="c", subcore_axis_name="s")
    sc_params = pltpu.CompilerParams(needs_layout_passes=False)

    dispatch = pl.kernel(
        _dispatch_body,
        out_type=jax.ShapeDtypeStruct((_EC + 8, _D // 2), jnp.int32),
        mesh=mesh,
        scratch_types=[
            pltpu.VMEM((_TOK_PER_W,), jnp.int32),
            pltpu.VMEM((_TOK_PER_W,), jnp.int32),
            pltpu.VMEM((_TOK_PER_W, _D // 2), jnp.int32),
            pltpu.SemaphoreType.DMA,
            pltpu.SemaphoreType.DMA,
            pltpu.SemaphoreType.DMA,
        ],
        compiler_params=sc_params,
    )
    eb = dispatch(sidx.reshape(2 * _T), x16)

    eo = pl.pallas_call(
        _mlp_body,
        grid=(_E, _HC),
        in_specs=[
            pl.BlockSpec((_CAP, _D // 2), lambda e, h: (e, 0)),
            pl.BlockSpec((1, _D, _HB), lambda e, h: (e, 0, h)),
            pl.BlockSpec((1, _HB, _D), lambda e, h: (e, h, 0)),
        ],
        out_specs=pl.BlockSpec((_CAP, _D), lambda e, h: (e, 0)),
        out_shape=jax.ShapeDtypeStruct((_EC, _D), jnp.float32),
    )(eb, w_fc, w_proj)

    combine = pl.kernel(
        _combine_body,
        out_type=jax.ShapeDtypeStruct((_T, _D), jnp.float32),
        mesh=mesh,
        scratch_types=[
            pltpu.VMEM((_TOK_PER_W,), jnp.int32),
            pltpu.VMEM((_TOK_PER_W,), jnp.int32),
            pltpu.VMEM((_TOK_PER_W,), jnp.float32),
            pltpu.VMEM((_TOK_PER_W,), jnp.float32),
            pltpu.VMEM((_TOK_PER_W, _D), jnp.float32),
            pltpu.VMEM((_TOK_PER_W, _D), jnp.float32),
            pltpu.SemaphoreType.DMA,
            pltpu.SemaphoreType.DMA,
        ],
        compiler_params=sc_params,
    )
    y = combine(cidx.reshape(2 * _T), wflat.reshape(2 * _T), eo)

    return y.reshape(1, _T, _D), lb[0, 0], zz[0, 0]


# speedup vs baseline: 1.0279x; 1.0279x over previous
import jax
import jax.numpy as jnp
from jax import lax
from jax.experimental import pallas as pl
from jax.experimental.pallas import tpu as pltpu
from jax.experimental.pallas import tpu_sc as plsc

_D = 768
_E = 8
_H = 3072
_T = 2048
_CAP = 640
_EC = _E * _CAP
_NW = 32
_ROWS_PER_W = _EC // _NW
_GCHUNK = 80
_TOK_PER_W = _T // _NW
_HC = 1
_HB = _H // _HC


def _router_body(x_ref, wg_ref, sidx_ref, cidx_ref, wflat_ref, lb_ref, zz_ref):
    x = x_ref[...]
    wg = wg_ref[...]
    logits = jnp.dot(x, wg, preferred_element_type=jnp.float32)

    ei = lax.broadcasted_iota(jnp.int32, (_T, _E), 1)
    m1 = jnp.max(logits, axis=1, keepdims=True)
    a1 = jnp.min(jnp.where(logits == m1, ei, _E), axis=1, keepdims=True)
    l2 = jnp.where(ei == a1, -jnp.inf, logits)
    m2 = jnp.max(l2, axis=1, keepdims=True)
    a2 = jnp.min(jnp.where(l2 == m2, ei, _E), axis=1, keepdims=True)

    d = jnp.exp(m2 - m1)
    w1 = 1.0 / (1.0 + d)
    w2 = d / (1.0 + d)

    oh1 = (ei == a1).astype(jnp.float32)
    oh2 = (ei == a2).astype(jnp.float32)
    oh = jnp.concatenate([oh1, oh2], axis=0)
    c = oh
    s = 1
    while s < 2 * _T:
        pad = jnp.zeros((s, _E), jnp.float32)
        c = c + jnp.concatenate([pad, c[: 2 * _T - s]], axis=0)
        s *= 2
    rank = jnp.sum(oh * c, axis=1, keepdims=True) - 1.0
    ranki = rank.astype(jnp.int32)

    e_flat = jnp.concatenate([a1, a2], axis=0)
    w_flat = jnp.concatenate([w1, w2], axis=0)
    valid = (ranki < _CAP) & (w_flat > 0.0)
    slot = e_flat * _CAP + ranki
    ji = lax.broadcasted_iota(jnp.int32, (2 * _T, 1), 0)
    sidx_ref[...] = jnp.where(valid, slot, _EC + (ji & 7))

    counts = c[2 * _T - 1 : 2 * _T, :]
    ei_row = lax.broadcasted_iota(jnp.int32, (1, _E), 1)
    cmax = jnp.max(counts, axis=1, keepdims=True)
    emax = jnp.min(jnp.where(counts == cmax, ei_row, _E), axis=1, keepdims=True)
    dummy = emax * _CAP
    cidx_ref[...] = jnp.where(valid, slot, dummy)
    wflat_ref[...] = jnp.where(valid, w_flat, 0.0)

    pe = jnp.exp(logits - m1)
    se = jnp.sum(pe, axis=1, keepdims=True)
    probs = pe / se
    ppe = jnp.sum(probs, axis=0, keepdims=True) * (1.0 / _T)
    v1 = valid[:_T].astype(jnp.float32)
    v2 = valid[_T:].astype(jnp.float32)
    tpe = jnp.sum(oh1 * v1 + oh2 * v2, axis=0, keepdims=True) * (1.0 / _T)
    lb_ref[...] = _E * jnp.sum(ppe * tpe, axis=1, keepdims=True)
    z = m1 + jnp.log(se)
    zz_ref[...] = jnp.sum(z * z, axis=0, keepdims=True) * (1.0 / _T)


def _dispatch_body(sidx_hbm, x_hbm, eb_hbm, i0_v, i1_v, rows_v, semg, sem0, sem1):
    cid = lax.axis_index("c")
    sid = lax.axis_index("s")
    wid = sid * 2 + cid
    base = wid * _TOK_PER_W

    g = pltpu.async_copy(x_hbm.at[pl.ds(base, _TOK_PER_W)], rows_v, semg)
    pltpu.sync_copy(sidx_hbm.at[pl.ds(base, _TOK_PER_W)], i0_v)
    pltpu.sync_copy(sidx_hbm.at[pl.ds(_T + base, _TOK_PER_W)], i1_v)
    g.wait()
    c0 = pltpu.async_copy(rows_v, eb_hbm.at[i0_v], sem0)
    c1 = pltpu.async_copy(rows_v, eb_hbm.at[i1_v], sem1)
    c0.wait()
    c1.wait()


def _mlp_body(eb_ref, wfc_ref, wp_ref, out_ref):
    hc = pl.program_id(1)
    h = jnp.dot(eb_ref[...].astype(jnp.bfloat16), wfc_ref[0].astype(jnp.bfloat16),
                preferred_element_type=jnp.float32)
    h = 0.5 * h * (1.0 + lax.erf(h * 0.7071067811865476))
    part = jnp.dot(h.astype(jnp.bfloat16), wp_ref[0].astype(jnp.bfloat16),
                   preferred_element_type=jnp.float32)

    @pl.when(hc == 0)
    def _():
        out_ref[...] = part

    @pl.when(hc > 0)
    def _():
        out_ref[...] = out_ref[...] + part


def _combine_body(cidx_hbm, wf_hbm, eo_hbm, y_hbm,
                  i0_v, i1_v, w0_v, w1_v, b0, b1, sem0, sem1):
    cid = lax.axis_index("c")
    sid = lax.axis_index("s")
    wid = sid * 2 + cid
    base = wid * _TOK_PER_W

    pltpu.sync_copy(cidx_hbm.at[pl.ds(base, _TOK_PER_W)], i0_v)
    pltpu.sync_copy(cidx_hbm.at[pl.ds(_T + base, _TOK_PER_W)], i1_v)
    pltpu.sync_copy(wf_hbm.at[pl.ds(base, _TOK_PER_W)], w0_v)
    pltpu.sync_copy(wf_hbm.at[pl.ds(_T + base, _TOK_PER_W)], w1_v)

    hw = _TOK_PER_W // 2
    copies = []
    for half in range(2):
        r0 = half * hw
        copies.append(pltpu.async_copy(
            eo_hbm.at[i0_v.at[pl.ds(r0, hw)]], b0.at[pl.ds(r0, hw)], sem0))
        copies.append(pltpu.async_copy(
            eo_hbm.at[i1_v.at[pl.ds(r0, hw)]], b1.at[pl.ds(r0, hw)], sem1))

    def addrow(r, carry):
        rv = jnp.zeros((16,), jnp.int32) + r
        w0 = plsc.load_gather(w0_v, [rv])
        w1 = plsc.load_gather(w1_v, [rv])
        for cc in range(_D // 16):
            sl = pl.ds(cc * 16, 16)
            b0[r, sl] = b0[r, sl] * w0 + b1[r, sl] * w1
        return carry

    for half in range(2):
        copies[2 * half].wait()
        copies[2 * half + 1].wait()
        r0 = half * hw
        lax.fori_loop(r0, r0 + hw, addrow, 0)
        pltpu.sync_copy(b0.at[pl.ds(r0, hw)],
                        y_hbm.at[pl.ds(base + r0, hw)])


def kernel(x, w_gate, w_fc, w_proj):
    xf = x.reshape(_T, _D)

    sidx, cidx, wflat, lb, zz = pl.pallas_call(
        _router_body,
        out_shape=[
            jax.ShapeDtypeStruct((2 * _T, 1), jnp.int32),
            jax.ShapeDtypeStruct((2 * _T, 1), jnp.int32),
            jax.ShapeDtypeStruct((2 * _T, 1), jnp.float32),
            jax.ShapeDtypeStruct((1, 1), jnp.float32),
            jax.ShapeDtypeStruct((1, 1), jnp.float32),
        ],
    )(xf, w_gate)

    mesh = plsc.VectorSubcoreMesh(core_axis_name="c", subcore_axis_name="s")
    sc_params = pltpu.CompilerParams(needs_layout_passes=False)

    dispatch = pl.kernel(
        _dispatch_body,
        out_type=jax.ShapeDtypeStruct((_EC + 8, _D), jnp.float32),
        mesh=mesh,
        scratch_types=[
            pltpu.VMEM((_TOK_PER_W,), jnp.int32),
            pltpu.VMEM((_TOK_PER_W,), jnp.int32),
            pltpu.VMEM((_TOK_PER_W, _D), jnp.float32),
            pltpu.SemaphoreType.DMA,
            pltpu.SemaphoreType.DMA,
            pltpu.SemaphoreType.DMA,
        ],
        compiler_params=sc_params,
    )
    eb = dispatch(sidx.reshape(2 * _T), xf)

    eo = pl.pallas_call(
        _mlp_body,
        grid=(_E, _HC),
        in_specs=[
            pl.BlockSpec((_CAP, _D), lambda e, h: (e, 0)),
            pl.BlockSpec((1, _D, _HB), lambda e, h: (e, 0, h)),
            pl.BlockSpec((1, _HB, _D), lambda e, h: (e, h, 0)),
        ],
        out_specs=pl.BlockSpec((_CAP, _D), lambda e, h: (e, 0)),
        out_shape=jax.ShapeDtypeStruct((_EC, _D), jnp.float32),
    )(eb, w_fc, w_proj)

    combine = pl.kernel(
        _combine_body,
        out_type=jax.ShapeDtypeStruct((_T, _D), jnp.float32),
        mesh=mesh,
        scratch_types=[
            pltpu.VMEM((_TOK_PER_W,), jnp.int32),
            pltpu.VMEM((_TOK_PER_W,), jnp.int32),
            pltpu.VMEM((_TOK_PER_W,), jnp.float32),
            pltpu.VMEM((_TOK_PER_W,), jnp.float32),
            pltpu.VMEM((_TOK_PER_W, _D), jnp.float32),
            pltpu.VMEM((_TOK_PER_W, _D), jnp.float32),
            pltpu.SemaphoreType.DMA,
            pltpu.SemaphoreType.DMA,
        ],
        compiler_params=sc_params,
    )
    y = combine(cidx.reshape(2 * _T), wflat.reshape(2 * _T), eo)

    return y.reshape(1, _T, _D), lb[0, 0], zz[0, 0]


# speedup vs baseline: 1.0281x; 1.0002x over previous
import jax
import jax.numpy as jnp
from jax import lax
from jax.experimental import pallas as pl
from jax.experimental.pallas import tpu as pltpu
from jax.experimental.pallas import tpu_sc as plsc

_D = 768
_E = 8
_H = 3072
_T = 2048
_CAP = 640
_EC = _E * _CAP
_NW = 32
_TOK_PER_W = _T // _NW
_HC = 1
_HB = _H // _HC


def _router_body(x_ref, wg_ref, sidx_ref, cidx_ref, wflat_ref, lb_ref, zz_ref):
    x = x_ref[...]
    wg = wg_ref[...]
    logits = jnp.dot(x, wg, preferred_element_type=jnp.float32)

    ei = lax.broadcasted_iota(jnp.int32, (_T, _E), 1)
    m1 = jnp.max(logits, axis=1, keepdims=True)
    a1 = jnp.min(jnp.where(logits == m1, ei, _E), axis=1, keepdims=True)
    l2 = jnp.where(ei == a1, -jnp.inf, logits)
    m2 = jnp.max(l2, axis=1, keepdims=True)
    a2 = jnp.min(jnp.where(l2 == m2, ei, _E), axis=1, keepdims=True)

    d = jnp.exp(m2 - m1)
    w1 = 1.0 / (1.0 + d)
    w2 = d / (1.0 + d)

    oh1 = (ei == a1).astype(jnp.float32)
    oh2 = (ei == a2).astype(jnp.float32)
    oh = jnp.concatenate([oh1, oh2], axis=0)
    c = oh
    s = 1
    while s < 2 * _T:
        pad = jnp.zeros((s, _E), jnp.float32)
        c = c + jnp.concatenate([pad, c[: 2 * _T - s]], axis=0)
        s *= 2
    rank = jnp.sum(oh * c, axis=1, keepdims=True) - 1.0
    ranki = rank.astype(jnp.int32)

    e_flat = jnp.concatenate([a1, a2], axis=0)
    w_flat = jnp.concatenate([w1, w2], axis=0)
    valid = (ranki < _CAP) & (w_flat > 0.0)
    slot = e_flat * _CAP + ranki
    ji = lax.broadcasted_iota(jnp.int32, (2 * _T, 1), 0)
    sidx_ref[...] = jnp.where(valid, slot, _EC + (ji & 7))

    counts = c[2 * _T - 1 : 2 * _T, :]
    ei_row = lax.broadcasted_iota(jnp.int32, (1, _E), 1)
    cmax = jnp.max(counts, axis=1, keepdims=True)
    emax = jnp.min(jnp.where(counts == cmax, ei_row, _E), axis=1, keepdims=True)
    dummy = emax * _CAP
    cidx_ref[...] = jnp.where(valid, slot, dummy)
    wflat_ref[...] = jnp.where(valid, w_flat, 0.0)

    pe = jnp.exp(logits - m1)
    se = jnp.sum(pe, axis=1, keepdims=True)
    probs = pe / se
    ppe = jnp.sum(probs, axis=0, keepdims=True) * (1.0 / _T)
    v1 = valid[:_T].astype(jnp.float32)
    v2 = valid[_T:].astype(jnp.float32)
    tpe = jnp.sum(oh1 * v1 + oh2 * v2, axis=0, keepdims=True) * (1.0 / _T)
    lb_ref[...] = _E * jnp.sum(ppe * tpe, axis=1, keepdims=True)
    z = m1 + jnp.log(se)
    zz_ref[...] = jnp.sum(z * z, axis=0, keepdims=True) * (1.0 / _T)


def _dispatch_body(sidx_hbm, x_hbm, eb_hbm, i0_v, i1_v, rows_v, semg, sem0, sem1):
    cid = lax.axis_index("c")
    sid = lax.axis_index("s")
    wid = sid * 2 + cid
    base = wid * _TOK_PER_W

    g = pltpu.async_copy(x_hbm.at[pl.ds(base, _TOK_PER_W)], rows_v, semg)
    pltpu.sync_copy(sidx_hbm.at[pl.ds(base, _TOK_PER_W)], i0_v)
    pltpu.sync_copy(sidx_hbm.at[pl.ds(_T + base, _TOK_PER_W)], i1_v)
    g.wait()
    c0 = pltpu.async_copy(rows_v, eb_hbm.at[i0_v], sem0)
    c1 = pltpu.async_copy(rows_v, eb_hbm.at[i1_v], sem1)
    c0.wait()
    c1.wait()


def _mlp_body(eb_ref, wfc_ref, wp_ref, out_ref):
    hc = pl.program_id(1)
    h = jnp.dot(eb_ref[...].astype(jnp.bfloat16), wfc_ref[0].astype(jnp.bfloat16),
                preferred_element_type=jnp.float32)
    h = 0.5 * h * (1.0 + lax.erf(h * 0.7071067811865476))
    part = jnp.dot(h.astype(jnp.bfloat16), wp_ref[0].astype(jnp.bfloat16),
                   preferred_element_type=jnp.float32)

    @pl.when(hc == 0)
    def _():
        out_ref[...] = part

    @pl.when(hc > 0)
    def _():
        out_ref[...] = out_ref[...] + part


def _combine_body(cidx_hbm, wf_hbm, eo_hbm, y_hbm,
                  i0_v, i1_v, w0_v, w1_v, b0, b1, sem0, sem1):
    cid = lax.axis_index("c")
    sid = lax.axis_index("s")
    wid = sid * 2 + cid
    base = wid * _TOK_PER_W

    pltpu.sync_copy(cidx_hbm.at[pl.ds(base, _TOK_PER_W)], i0_v)
    pltpu.sync_copy(cidx_hbm.at[pl.ds(_T + base, _TOK_PER_W)], i1_v)
    pltpu.sync_copy(wf_hbm.at[pl.ds(base, _TOK_PER_W)], w0_v)
    pltpu.sync_copy(wf_hbm.at[pl.ds(_T + base, _TOK_PER_W)], w1_v)

    hw = _TOK_PER_W // 2
    copies = []
    for half in range(2):
        r0 = half * hw
        copies.append(pltpu.async_copy(
            eo_hbm.at[i0_v.at[pl.ds(r0, hw)]], b0.at[pl.ds(r0, hw)], sem0))
        copies.append(pltpu.async_copy(
            eo_hbm.at[i1_v.at[pl.ds(r0, hw)]], b1.at[pl.ds(r0, hw)], sem1))

    def addrow(r, carry):
        rv = jnp.zeros((16,), jnp.int32) + r
        w0 = plsc.load_gather(w0_v, [rv])
        w1 = plsc.load_gather(w1_v, [rv])
        for cc in range(_D // 16):
            sl = pl.ds(cc * 16, 16)
            b0[r, sl] = b0[r, sl] * w0 + b1[r, sl] * w1
        return carry

    for half in range(2):
        copies[2 * half].wait()
        copies[2 * half + 1].wait()
        r0 = half * hw
        lax.fori_loop(r0, r0 + hw, addrow, 0)
        pltpu.sync_copy(b0.at[pl.ds(r0, hw)],
                        y_hbm.at[pl.ds(base + r0, hw)])


def kernel(x, w_gate, w_fc, w_proj):
    xf = x.reshape(_T, _D)

    sidx, cidx, wflat, lb, zz = pl.pallas_call(
        _router_body,
        out_shape=[
            jax.ShapeDtypeStruct((2 * _T, 1), jnp.int32),
            jax.ShapeDtypeStruct((2 * _T, 1), jnp.int32),
            jax.ShapeDtypeStruct((2 * _T, 1), jnp.float32),
            jax.ShapeDtypeStruct((1, 1), jnp.float32),
            jax.ShapeDtypeStruct((1, 1), jnp.float32),
        ],
    )(xf, w_gate)

    mesh = plsc.VectorSubcoreMesh(core_axis_name="c", subcore_axis_name="s")
    sc_params = pltpu.CompilerParams(needs_layout_passes=False)

    dispatch = pl.kernel(
        _dispatch_body,
        out_type=jax.ShapeDtypeStruct((_EC + 8, _D), jnp.float32),
        mesh=mesh,
        scratch_types=[
            pltpu.VMEM((_TOK_PER_W,), jnp.int32),
            pltpu.VMEM((_TOK_PER_W,), jnp.int32),
            pltpu.VMEM((_TOK_PER_W, _D), jnp.float32),
            pltpu.SemaphoreType.DMA,
            pltpu.SemaphoreType.DMA,
            pltpu.SemaphoreType.DMA,
        ],
        compiler_params=sc_params,
    )
    eb = dispatch(sidx.reshape(2 * _T), xf)

    eo = pl.pallas_call(
        _mlp_body,
        grid=(_E, _HC),
        in_specs=[
            pl.BlockSpec((_CAP, _D), lambda e, h: (e, 0)),
            pl.BlockSpec((1, _D, _HB), lambda e, h: (e, 0, h)),
            pl.BlockSpec((1, _HB, _D), lambda e, h: (e, h, 0)),
        ],
        out_specs=pl.BlockSpec((_CAP, _D), lambda e, h: (e, 0)),
        out_shape=jax.ShapeDtypeStruct((_EC, _D), jnp.float32),
    )(eb, w_fc, w_proj)

    combine = pl.kernel(
        _combine_body,
        out_type=jax.ShapeDtypeStruct((_T, _D), jnp.float32),
        mesh=mesh,
        scratch_types=[
            pltpu.VMEM((_TOK_PER_W,), jnp.int32),
            pltpu.VMEM((_TOK_PER_W,), jnp.int32),
            pltpu.VMEM((_TOK_PER_W,), jnp.float32),
            pltpu.VMEM((_TOK_PER_W,), jnp.float32),
            pltpu.VMEM((_TOK_PER_W, _D), jnp.float32),
            pltpu.VMEM((_TOK_PER_W, _D), jnp.float32),
            pltpu.SemaphoreType.DMA,
            pltpu.SemaphoreType.DMA,
        ],
        compiler_params=sc_params,
    )
    y = combine(cidx.reshape(2 * _T), wflat.reshape(2 * _T), eo)

    return y.reshape(1, _T, _D), lb[0, 0], zz[0, 0]


# speedup vs baseline: 1.0302x; 1.0020x over previous
import jax
import jax.numpy as jnp
from jax import lax
from jax.experimental import pallas as pl
from jax.experimental.pallas import tpu as pltpu
from jax.experimental.pallas import tpu_sc as plsc

_D = 768
_E = 8
_H = 3072
_T = 2048
_CAP = 640
_EC = _E * _CAP
_NW = 32
_TOK_PER_W = _T // _NW
_HC = 1
_HB = _H // _HC


def _router_body(x_ref, wg_ref, sidx_ref, cidx_ref, wflat_ref, lb_ref, zz_ref):
    x = x_ref[...]
    wg = wg_ref[...]
    logits = jnp.dot(x, wg, preferred_element_type=jnp.float32)

    ei = lax.broadcasted_iota(jnp.int32, (_T, _E), 1)
    m1 = jnp.max(logits, axis=1, keepdims=True)
    a1 = jnp.min(jnp.where(logits == m1, ei, _E), axis=1, keepdims=True)
    l2 = jnp.where(ei == a1, -jnp.inf, logits)
    m2 = jnp.max(l2, axis=1, keepdims=True)
    a2 = jnp.min(jnp.where(l2 == m2, ei, _E), axis=1, keepdims=True)

    d = jnp.exp(m2 - m1)
    w1 = 1.0 / (1.0 + d)
    w2 = d / (1.0 + d)

    oh1 = (ei == a1).astype(jnp.float32)
    oh2 = (ei == a2).astype(jnp.float32)
    oh = jnp.concatenate([oh1, oh2], axis=0)
    c = oh
    s = 1
    while s < 2 * _T:
        pad = jnp.zeros((s, _E), jnp.float32)
        c = c + jnp.concatenate([pad, c[: 2 * _T - s]], axis=0)
        s *= 2
    rank = jnp.sum(oh * c, axis=1, keepdims=True) - 1.0
    ranki = rank.astype(jnp.int32)

    e_flat = jnp.concatenate([a1, a2], axis=0)
    w_flat = jnp.concatenate([w1, w2], axis=0)
    valid = (ranki < _CAP) & (w_flat > 0.0)
    slot = e_flat * _CAP + ranki
    ji = lax.broadcasted_iota(jnp.int32, (2 * _T, 1), 0)
    sidx_ref[...] = jnp.where(valid, slot, _EC + (ji & 7))

    counts = c[2 * _T - 1 : 2 * _T, :]
    ei_row = lax.broadcasted_iota(jnp.int32, (1, _E), 1)
    cmax = jnp.max(counts, axis=1, keepdims=True)
    emax = jnp.min(jnp.where(counts == cmax, ei_row, _E), axis=1, keepdims=True)
    dummy = emax * _CAP
    cidx_ref[...] = jnp.where(valid, slot, dummy)
    wflat_ref[...] = jnp.where(valid, w_flat, 0.0)

    pe = jnp.exp(logits - m1)
    se = jnp.sum(pe, axis=1, keepdims=True)
    probs = pe / se
    ppe = jnp.sum(probs, axis=0, keepdims=True) * (1.0 / _T)
    v1 = valid[:_T].astype(jnp.float32)
    v2 = valid[_T:].astype(jnp.float32)
    tpe = jnp.sum(oh1 * v1 + oh2 * v2, axis=0, keepdims=True) * (1.0 / _T)
    lb_ref[...] = _E * jnp.sum(ppe * tpe, axis=1, keepdims=True)
    z = m1 + jnp.log(se)
    zz_ref[...] = jnp.sum(z * z, axis=0, keepdims=True) * (1.0 / _T)


def _dispatch_body(sidx_hbm, x_hbm, eb_hbm, i0_v, i1_v, rows_v, semg, sem0, sem1):
    cid = lax.axis_index("c")
    sid = lax.axis_index("s")
    wid = sid * 2 + cid
    base = wid * _TOK_PER_W

    g = pltpu.async_copy(x_hbm.at[pl.ds(base, _TOK_PER_W)], rows_v, semg)
    pltpu.sync_copy(sidx_hbm.at[pl.ds(base, _TOK_PER_W)], i0_v)
    pltpu.sync_copy(sidx_hbm.at[pl.ds(_T + base, _TOK_PER_W)], i1_v)
    g.wait()
    c0 = pltpu.async_copy(rows_v, eb_hbm.at[i0_v], sem0)
    c1 = pltpu.async_copy(rows_v, eb_hbm.at[i1_v], sem1)
    c0.wait()
    c1.wait()


def _mlp_body(eb_ref, wfc_ref, wp_ref, out_ref):
    hc = pl.program_id(1)
    h = jnp.dot(eb_ref[...].astype(jnp.bfloat16), wfc_ref[0].astype(jnp.bfloat16),
                preferred_element_type=jnp.float32)
    h = 0.5 * h * (1.0 + lax.erf(h * 0.7071067811865476))
    part = jnp.dot(h.astype(jnp.bfloat16), wp_ref[0].astype(jnp.bfloat16),
                   preferred_element_type=jnp.float32)

    @pl.when(hc == 0)
    def _():
        out_ref[...] = part

    @pl.when(hc > 0)
    def _():
        out_ref[...] = out_ref[...] + part


def _combine_body(cidx_hbm, wf_hbm, eo_hbm, y_hbm,
                  i0_v, i1_v, w0_v, w1_v, b0, b1, sem0, sem1):
    cid = lax.axis_index("c")
    sid = lax.axis_index("s")
    wid = sid * 2 + cid
    base = wid * _TOK_PER_W

    pltpu.sync_copy(cidx_hbm.at[pl.ds(base, _TOK_PER_W)], i0_v)
    pltpu.sync_copy(cidx_hbm.at[pl.ds(_T + base, _TOK_PER_W)], i1_v)
    pltpu.sync_copy(wf_hbm.at[pl.ds(base, _TOK_PER_W)], w0_v)
    pltpu.sync_copy(wf_hbm.at[pl.ds(_T + base, _TOK_PER_W)], w1_v)

    nq = 4
    hw = _TOK_PER_W // nq
    copies = []
    for q in range(nq):
        r0 = q * hw
        copies.append(pltpu.async_copy(
            eo_hbm.at[i0_v.at[pl.ds(r0, hw)]], b0.at[pl.ds(r0, hw)], sem0))
        copies.append(pltpu.async_copy(
            eo_hbm.at[i1_v.at[pl.ds(r0, hw)]], b1.at[pl.ds(r0, hw)], sem1))

    def addrow(r, carry):
        rv = jnp.zeros((16,), jnp.int32) + r
        w0 = plsc.load_gather(w0_v, [rv])
        w1 = plsc.load_gather(w1_v, [rv])
        for cc in range(_D // 16):
            sl = pl.ds(cc * 16, 16)
            b0[r, sl] = b0[r, sl] * w0 + b1[r, sl] * w1
        return carry

    for q in range(nq):
        copies[2 * q].wait()
        copies[2 * q + 1].wait()
        r0 = q * hw
        lax.fori_loop(r0, r0 + hw, addrow, 0)
        pltpu.sync_copy(b0.at[pl.ds(r0, hw)],
                        y_hbm.at[pl.ds(base + r0, hw)])


def kernel(x, w_gate, w_fc, w_proj):
    xf = x.reshape(_T, _D)

    sidx, cidx, wflat, lb, zz = pl.pallas_call(
        _router_body,
        out_shape=[
            jax.ShapeDtypeStruct((2 * _T, 1), jnp.int32),
            jax.ShapeDtypeStruct((2 * _T, 1), jnp.int32),
            jax.ShapeDtypeStruct((2 * _T, 1), jnp.float32),
            jax.ShapeDtypeStruct((1, 1), jnp.float32),
            jax.ShapeDtypeStruct((1, 1), jnp.float32),
        ],
    )(xf, w_gate)

    mesh = plsc.VectorSubcoreMesh(core_axis_name="c", subcore_axis_name="s")
    sc_params = pltpu.CompilerParams(needs_layout_passes=False)

    dispatch = pl.kernel(
        _dispatch_body,
        out_type=jax.ShapeDtypeStruct((_EC + 8, _D), jnp.float32),
        mesh=mesh,
        scratch_types=[
            pltpu.VMEM((_TOK_PER_W,), jnp.int32),
            pltpu.VMEM((_TOK_PER_W,), jnp.int32),
            pltpu.VMEM((_TOK_PER_W, _D), jnp.float32),
            pltpu.SemaphoreType.DMA,
            pltpu.SemaphoreType.DMA,
            pltpu.SemaphoreType.DMA,
        ],
        compiler_params=sc_params,
    )
    eb = dispatch(sidx.reshape(2 * _T), xf)

    eo = pl.pallas_call(
        _mlp_body,
        grid=(_E, _HC),
        in_specs=[
            pl.BlockSpec((_CAP, _D), lambda e, h: (e, 0)),
            pl.BlockSpec((1, _D, _HB), lambda e, h: (e, 0, h)),
            pl.BlockSpec((1, _HB, _D), lambda e, h: (e, h, 0)),
        ],
        out_specs=pl.BlockSpec((_CAP, _D), lambda e, h: (e, 0)),
        out_shape=jax.ShapeDtypeStruct((_EC, _D), jnp.float32),
    )(eb, w_fc, w_proj)

    combine = pl.kernel(
        _combine_body,
        out_type=jax.ShapeDtypeStruct((_T, _D), jnp.float32),
        mesh=mesh,
        scratch_types=[
            pltpu.VMEM((_TOK_PER_W,), jnp.int32),
            pltpu.VMEM((_TOK_PER_W,), jnp.int32),
            pltpu.VMEM((_TOK_PER_W,), jnp.float32),
            pltpu.VMEM((_TOK_PER_W,), jnp.float32),
            pltpu.VMEM((_TOK_PER_W, _D), jnp.float32),
            pltpu.VMEM((_TOK_PER_W, _D), jnp.float32),
            pltpu.SemaphoreType.DMA,
            pltpu.SemaphoreType.DMA,
        ],
        compiler_params=sc_params,
    )
    y = combine(cidx.reshape(2 * _T), wflat.reshape(2 * _T), eo)

    return y.reshape(1, _T, _D), lb[0, 0], zz[0, 0]


# speedup vs baseline: 1.0864x; 1.0546x over previous
import jax
import jax.numpy as jnp
from jax import lax
from jax.experimental import pallas as pl
from jax.experimental.pallas import tpu as pltpu
from jax.experimental.pallas import tpu_sc as plsc

_D = 768
_E = 8
_H = 3072
_T = 2048
_CAP = 640
_EC = _E * _CAP
_NW = 32
_TOK_PER_W = _T // _NW
_HC = 1
_HB = _H // _HC


def _router_body(x_ref, wg_ref, sidx_ref, cidx_ref, wflat_ref, lb_ref, zz_ref):
    x = x_ref[...]
    wg = wg_ref[...]
    lg = jnp.dot(x, wg, preferred_element_type=jnp.float32).T

    ei = lax.broadcasted_iota(jnp.int32, (_E, _T), 0)
    m1 = jnp.max(lg, axis=0, keepdims=True)
    a1 = jnp.min(jnp.where(lg == m1, ei, _E), axis=0, keepdims=True)
    l2 = jnp.where(ei == a1, -jnp.inf, lg)
    m2 = jnp.max(l2, axis=0, keepdims=True)
    a2 = jnp.min(jnp.where(l2 == m2, ei, _E), axis=0, keepdims=True)

    d = jnp.exp(m2 - m1)
    w1 = 1.0 / (1.0 + d)
    w2 = d / (1.0 + d)

    oh1 = (ei == a1).astype(jnp.float32)
    oh2 = (ei == a2).astype(jnp.float32)
    c = jnp.concatenate([oh1, oh2], axis=0)
    s = 1
    while s < _T:
        pad = jnp.zeros((2 * _E, s), jnp.float32)
        c = c + jnp.concatenate([pad, c[:, : _T - s]], axis=1)
        s *= 2
    c1 = c[:_E]
    c2 = c[_E:]
    tot1 = c1[:, _T - 1 : _T]
    rank1 = jnp.sum(oh1 * c1, axis=0, keepdims=True) - 1.0
    rank2 = jnp.sum(oh2 * (c2 + tot1), axis=0, keepdims=True) - 1.0
    r1i = rank1.astype(jnp.int32)
    r2i = rank2.astype(jnp.int32)

    valid1 = (r1i < _CAP) & (w1 > 0.0)
    valid2 = (r2i < _CAP) & (w2 > 0.0)
    slot1 = a1 * _CAP + r1i
    slot2 = a2 * _CAP + r2i
    ji = lax.broadcasted_iota(jnp.int32, (1, _T), 1)
    pad_idx = _EC + (ji & 7)
    sidx_ref[...] = jnp.concatenate(
        [jnp.where(valid1, slot1, pad_idx),
         jnp.where(valid2, slot2, pad_idx)], axis=0)

    counts = tot1 + c2[:, _T - 1 : _T]
    ei_col = lax.broadcasted_iota(jnp.int32, (_E, 1), 0)
    cmax = jnp.max(counts, axis=0, keepdims=True)
    emax = jnp.min(jnp.where(counts == cmax, ei_col, _E), axis=0, keepdims=True)
    dummy = emax * _CAP
    cidx_ref[...] = jnp.concatenate(
        [jnp.where(valid1, slot1, dummy),
         jnp.where(valid2, slot2, dummy)], axis=0)
    wflat_ref[...] = jnp.concatenate(
        [jnp.where(valid1, w1, 0.0),
         jnp.where(valid2, w2, 0.0)], axis=0)

    pe = jnp.exp(lg - m1)
    se = jnp.sum(pe, axis=0, keepdims=True)
    probs = pe / se
    ppe = jnp.sum(probs, axis=1, keepdims=True) * (1.0 / _T)
    tpe = jnp.sum(oh1 * valid1.astype(jnp.float32)
                  + oh2 * valid2.astype(jnp.float32),
                  axis=1, keepdims=True) * (1.0 / _T)
    lb_ref[...] = _E * jnp.sum(ppe * tpe, axis=0, keepdims=True)
    z = m1 + jnp.log(se)
    zz_ref[...] = jnp.sum(z * z, axis=1, keepdims=True) * (1.0 / _T)


def _dispatch_body(sidx_hbm, x_hbm, eb_hbm, i0_v, i1_v, rows_v, semg, sem0, sem1):
    cid = lax.axis_index("c")
    sid = lax.axis_index("s")
    wid = sid * 2 + cid
    base = wid * _TOK_PER_W

    g = pltpu.async_copy(x_hbm.at[pl.ds(base, _TOK_PER_W)], rows_v, semg)
    pltpu.sync_copy(sidx_hbm.at[pl.ds(base, _TOK_PER_W)], i0_v)
    pltpu.sync_copy(sidx_hbm.at[pl.ds(_T + base, _TOK_PER_W)], i1_v)
    g.wait()
    c0 = pltpu.async_copy(rows_v, eb_hbm.at[i0_v], sem0)
    c1 = pltpu.async_copy(rows_v, eb_hbm.at[i1_v], sem1)
    c0.wait()
    c1.wait()


def _mlp_body(eb_ref, wfc_ref, wp_ref, out_ref):
    hc = pl.program_id(1)
    h = jnp.dot(eb_ref[...].astype(jnp.bfloat16), wfc_ref[0].astype(jnp.bfloat16),
                preferred_element_type=jnp.float32)
    h = 0.5 * h * (1.0 + lax.erf(h * 0.7071067811865476))
    part = jnp.dot(h.astype(jnp.bfloat16), wp_ref[0].astype(jnp.bfloat16),
                   preferred_element_type=jnp.float32)

    @pl.when(hc == 0)
    def _():
        out_ref[...] = part

    @pl.when(hc > 0)
    def _():
        out_ref[...] = out_ref[...] + part


def _combine_body(cidx_hbm, wf_hbm, eo_hbm, y_hbm,
                  i0_v, i1_v, w0_v, w1_v, b0, b1, sem0, sem1):
    cid = lax.axis_index("c")
    sid = lax.axis_index("s")
    wid = sid * 2 + cid
    base = wid * _TOK_PER_W

    pltpu.sync_copy(cidx_hbm.at[pl.ds(base, _TOK_PER_W)], i0_v)
    pltpu.sync_copy(cidx_hbm.at[pl.ds(_T + base, _TOK_PER_W)], i1_v)
    pltpu.sync_copy(wf_hbm.at[pl.ds(base, _TOK_PER_W)], w0_v)
    pltpu.sync_copy(wf_hbm.at[pl.ds(_T + base, _TOK_PER_W)], w1_v)

    nq = 4
    hw = _TOK_PER_W // nq
    copies = []
    for q in range(nq):
        r0 = q * hw
        copies.append(pltpu.async_copy(
            eo_hbm.at[i0_v.at[pl.ds(r0, hw)]], b0.at[pl.ds(r0, hw)], sem0))
        copies.append(pltpu.async_copy(
            eo_hbm.at[i1_v.at[pl.ds(r0, hw)]], b1.at[pl.ds(r0, hw)], sem1))

    def addrow(r, carry):
        rv = jnp.zeros((16,), jnp.int32) + r
        w0 = plsc.load_gather(w0_v, [rv])
        w1 = plsc.load_gather(w1_v, [rv])
        for cc in range(_D // 16):
            sl = pl.ds(cc * 16, 16)
            b0[r, sl] = b0[r, sl] * w0 + b1[r, sl] * w1
        return carry

    for q in range(nq):
        copies[2 * q].wait()
        copies[2 * q + 1].wait()
        r0 = q * hw
        lax.fori_loop(r0, r0 + hw, addrow, 0)
        pltpu.sync_copy(b0.at[pl.ds(r0, hw)],
                        y_hbm.at[pl.ds(base + r0, hw)])


def kernel(x, w_gate, w_fc, w_proj):
    xf = x.reshape(_T, _D)

    sidx, cidx, wflat, lb, zz = pl.pallas_call(
        _router_body,
        out_shape=[
            jax.ShapeDtypeStruct((2, _T), jnp.int32),
            jax.ShapeDtypeStruct((2, _T), jnp.int32),
            jax.ShapeDtypeStruct((2, _T), jnp.float32),
            jax.ShapeDtypeStruct((1, 1), jnp.float32),
            jax.ShapeDtypeStruct((1, 1), jnp.float32),
        ],
    )(xf, w_gate)

    mesh = plsc.VectorSubcoreMesh(core_axis_name="c", subcore_axis_name="s")
    sc_params = pltpu.CompilerParams(needs_layout_passes=False)

    dispatch = pl.kernel(
        _dispatch_body,
        out_type=jax.ShapeDtypeStruct((_EC + 8, _D), jnp.float32),
        mesh=mesh,
        scratch_types=[
            pltpu.VMEM((_TOK_PER_W,), jnp.int32),
            pltpu.VMEM((_TOK_PER_W,), jnp.int32),
            pltpu.VMEM((_TOK_PER_W, _D), jnp.float32),
            pltpu.SemaphoreType.DMA,
            pltpu.SemaphoreType.DMA,
            pltpu.SemaphoreType.DMA,
        ],
        compiler_params=sc_params,
    )
    eb = dispatch(sidx.reshape(2 * _T), xf)

    eo = pl.pallas_call(
        _mlp_body,
        grid=(_E, _HC),
        in_specs=[
            pl.BlockSpec((_CAP, _D), lambda e, h: (e, 0)),
            pl.BlockSpec((1, _D, _HB), lambda e, h: (e, 0, h)),
            pl.BlockSpec((1, _HB, _D), lambda e, h: (e, h, 0)),
        ],
        out_specs=pl.BlockSpec((_CAP, _D), lambda e, h: (e, 0)),
        out_shape=jax.ShapeDtypeStruct((_EC, _D), jnp.float32),
    )(eb, w_fc, w_proj)

    combine = pl.kernel(
        _combine_body,
        out_type=jax.ShapeDtypeStruct((_T, _D), jnp.float32),
        mesh=mesh,
        scratch_types=[
            pltpu.VMEM((_TOK_PER_W,), jnp.int32),
            pltpu.VMEM((_TOK_PER_W,), jnp.int32),
            pltpu.VMEM((_TOK_PER_W,), jnp.float32),
            pltpu.VMEM((_TOK_PER_W,), jnp.float32),
            pltpu.VMEM((_TOK_PER_W, _D), jnp.float32),
            pltpu.VMEM((_TOK_PER_W, _D), jnp.float32),
            pltpu.SemaphoreType.DMA,
            pltpu.SemaphoreType.DMA,
        ],
        compiler_params=sc_params,
    )
    y = combine(cidx.reshape(2 * _T), wflat.reshape(2 * _T), eo)

    return y.reshape(1, _T, _D), lb[0, 0], zz[0, 0]
